# initial kernel scaffold (unmeasured)
import jax
import jax.numpy as jnp
from jax import lax
from jax.experimental import pallas as pl
from jax.experimental.pallas import tpu as pltpu

N_DEV = 4
M = 2048
D = 2048
C = M // N_DEV


def kernel(partial, resid, gamma):
    x = partial.reshape(M, D)
    gamma2 = gamma.reshape(1, D)

    def body(x_ref, resid_ref, gamma_ref, out_ref,
             rs_ref, ag_ref, rs_send, rs_recv, ag_send, ag_recv):
        p = lax.axis_index("i")
        left = lax.rem(p + N_DEV - 1, N_DEV)
        right = lax.rem(p + 1, N_DEV)

        barrier = pltpu.get_barrier_semaphore()
        for nbr in (left, right):
            pl.semaphore_signal(barrier, inc=1, device_id=(nbr,),
                                device_id_type=pl.DeviceIdType.MESH)
        pl.semaphore_wait(barrier, 2)

        rs_ref[0] = x_ref[pl.ds(left * C, C), :].astype(jnp.bfloat16)
        for h in range(N_DEV - 1):
            s, r = h % 2, (h + 1) % 2
            rdma = pltpu.make_async_remote_copy(
                src_ref=rs_ref.at[s], dst_ref=rs_ref.at[r],
                send_sem=rs_send.at[s], recv_sem=rs_recv.at[r],
                device_id=(right,), device_id_type=pl.DeviceIdType.MESH)
            rdma.start()
            rdma.wait()
            if h < N_DEV - 2:
                ck = lax.rem(p + 2 - h, N_DEV)
                rs_ref[r] = rs_ref[r] + x_ref[pl.ds(ck * C, C), :].astype(
                    jnp.bfloat16)

        y = (rs_ref[(N_DEV - 1) % 2].astype(jnp.float32)
             + x_ref[pl.ds(p * C, C), :]
             + resid_ref[pl.ds(p * C, C), :])
        rms = jnp.sqrt(jnp.mean(y * y, axis=-1, keepdims=True) + 1e-6)
        o = y / rms * gamma_ref[:, :]
        out_ref[pl.ds(p * C, C), :] = o
        ag_ref[0] = o.astype(jnp.bfloat16)

        for h in range(N_DEV - 1):
            s, r = h % 2, (h + 1) % 2
            rdma = pltpu.make_async_remote_copy(
                src_ref=ag_ref.at[s], dst_ref=ag_ref.at[r],
                send_sem=ag_send.at[s], recv_sem=ag_recv.at[r],
                device_id=(right,), device_id_type=pl.DeviceIdType.MESH)
            rdma.start()
            rdma.wait()
            origin = lax.rem(p + N_DEV - 1 - h, N_DEV)
            out_ref[pl.ds(origin * C, C), :] = ag_ref[r].astype(jnp.float32)

    return pl.pallas_call(
        body,
        out_shape=jax.ShapeDtypeStruct((M, D), jnp.float32),
        in_specs=[
            pl.BlockSpec(memory_space=pltpu.VMEM),
            pl.BlockSpec(memory_space=pltpu.VMEM),
            pl.BlockSpec(memory_space=pltpu.VMEM),
        ],
        out_specs=pl.BlockSpec(memory_space=pltpu.VMEM),
        scratch_shapes=[
            pltpu.VMEM((2, C, D), jnp.bfloat16),
            pltpu.VMEM((2, C, D), jnp.bfloat16),
            pltpu.SemaphoreType.DMA((2,)),
            pltpu.SemaphoreType.DMA((2,)),
            pltpu.SemaphoreType.DMA((2,)),
            pltpu.SemaphoreType.DMA((2,)),
        ],
        compiler_params=pltpu.CompilerParams(collective_id=0),
    )(x, resid, gamma2)


# baseline (device time: 179862 ns/iter reference)
import jax
import jax.numpy as jnp
from jax import lax
from jax.experimental import pallas as pl
from jax.experimental.pallas import tpu as pltpu

N_DEV = 4
M = 2048
D = 2048
C = M // N_DEV


def kernel(partial, resid, gamma):
    x = partial.reshape(M, D)
    gamma2 = gamma.reshape(1, D)

    def body(x_ref, resid_ref, gamma_ref, out_ref,
             rs_ref, ag_ref, rs_send, rs_recv, ag_send, ag_recv):
        p = lax.axis_index("i")
        left = lax.rem(p + N_DEV - 1, N_DEV)
        right = lax.rem(p + 1, N_DEV)

        barrier = pltpu.get_barrier_semaphore()
        for nbr in (left, right):
            pl.semaphore_signal(barrier, inc=1, device_id=(nbr,),
                                device_id_type=pl.DeviceIdType.MESH)
        pl.semaphore_wait(barrier, 2)

        rs_ref[0] = x_ref[pl.ds(left * C, C), :].astype(jnp.bfloat16)
        for h in range(N_DEV - 1):
            s, r = h % 2, (h + 1) % 2
            rdma = pltpu.make_async_remote_copy(
                src_ref=rs_ref.at[s], dst_ref=rs_ref.at[r],
                send_sem=rs_send.at[s], recv_sem=rs_recv.at[r],
                device_id=(right,), device_id_type=pl.DeviceIdType.MESH)
            rdma.start()
            rdma.wait()
            if h < N_DEV - 2:
                ck = lax.rem(p + 2 - h, N_DEV)
                rs_ref[r] = rs_ref[r] + x_ref[pl.ds(ck * C, C), :].astype(
                    jnp.bfloat16)

        y = (rs_ref[(N_DEV - 1) % 2].astype(jnp.float32)
             + x_ref[pl.ds(p * C, C), :]
             + resid_ref[pl.ds(p * C, C), :])
        rms = jnp.sqrt(jnp.mean(y * y, axis=-1, keepdims=True) + 1e-6)
        o = y / rms * gamma_ref[:, :]
        out_ref[pl.ds(p * C, C), :] = o
        ag_ref[0] = o.astype(jnp.bfloat16)

        for h in range(N_DEV - 1):
            s, r = h % 2, (h + 1) % 2
            rdma = pltpu.make_async_remote_copy(
                src_ref=ag_ref.at[s], dst_ref=ag_ref.at[r],
                send_sem=ag_send.at[s], recv_sem=ag_recv.at[r],
                device_id=(right,), device_id_type=pl.DeviceIdType.MESH)
            rdma.start()
            rdma.wait()
            origin = lax.rem(p + N_DEV - 1 - h, N_DEV)
            out_ref[pl.ds(origin * C, C), :] = ag_ref[r].astype(jnp.float32)

    return pl.pallas_call(
        body,
        out_shape=jax.ShapeDtypeStruct((M, D), jnp.float32),
        in_specs=[
            pl.BlockSpec(memory_space=pltpu.VMEM),
            pl.BlockSpec(memory_space=pltpu.VMEM),
            pl.BlockSpec(memory_space=pltpu.VMEM),
        ],
        out_specs=pl.BlockSpec(memory_space=pltpu.VMEM),
        scratch_shapes=[
            pltpu.VMEM((2, C, D), jnp.bfloat16),
            pltpu.VMEM((2, C, D), jnp.bfloat16),
            pltpu.SemaphoreType.DMA((2,)),
            pltpu.SemaphoreType.DMA((2,)),
            pltpu.SemaphoreType.DMA((2,)),
            pltpu.SemaphoreType.DMA((2,)),
        ],
        compiler_params=pltpu.CompilerParams(
            collective_id=0, vmem_limit_bytes=100 * 1024 * 1024),
    )(x, resid, gamma2)


# device time: 100579 ns/iter; 1.7883x vs baseline; 1.7883x over previous
import jax
import jax.numpy as jnp
from jax import lax
from jax.experimental import pallas as pl
from jax.experimental.pallas import tpu as pltpu

N_DEV = 4
M = 2048
D = 2048
C = M // N_DEV
H = C // 2
LANES = 2
S = H // LANES


def kernel(partial, resid, gamma):
    x = partial.reshape(M, D)
    gamma2 = gamma.reshape(1, D)

    def body(x_ref, resid_ref, gamma_ref, out_ref,
             rsR, rsL, agR, agL,
             rsR_s, rsR_r, rsL_s, rsL_r,
             agR_s, agR_r, agL_s, agL_r):
        p = lax.axis_index("i")
        left = lax.rem(p + N_DEV - 1, N_DEV)
        right = lax.rem(p + 1, N_DEV)

        rs_buf = (rsR, rsL)
        ag_buf = (agR, agL)
        rs_sem = ((rsR_s, rsR_r), (rsL_s, rsL_r))
        ag_sem = ((agR_s, agR_r), (agL_s, agL_r))
        tgt = (right, left)
        base = (0, H)

        def send_chunk(d, h):
            return lax.rem(p + 3 - h, N_DEV) if d == 0 else \
                lax.rem(p + 1 + h, N_DEV)

        def recv_chunk(d, h):
            return lax.rem(p + 2 - h, N_DEV) if d == 0 else \
                lax.rem(p + 2 + h, N_DEV)

        def ag_origin(d, h):
            return lax.rem(p + 3 - h, N_DEV) if d == 0 else \
                lax.rem(p + 1 + h, N_DEV)

        def rows(ck, d, l):
            return pl.ds(ck * C + base[d] + l * S, S)

        def make_rdma(buf, sems, d, l, h):
            s_, r_ = h % 2, (h + 1) % 2
            return pltpu.make_async_remote_copy(
                src_ref=buf[d].at[s_, l], dst_ref=buf[d].at[r_, l],
                send_sem=sems[d][0].at[s_, l], recv_sem=sems[d][1].at[r_, l],
                device_id=(tgt[d],), device_id_type=pl.DeviceIdType.MESH)

        barrier = pltpu.get_barrier_semaphore()
        for nbr in (left, right):
            pl.semaphore_signal(barrier, inc=1, device_id=(nbr,),
                                device_id_type=pl.DeviceIdType.MESH)
        pl.semaphore_wait(barrier, 2)

        rs_fly = {}
        ag_fly = {}
        for l in range(LANES):
            for d in (0, 1):
                ck = send_chunk(d, 0)
                rs_buf[d][0, l] = x_ref[rows(ck, d, l), :].astype(jnp.bfloat16)
                rd = make_rdma(rs_buf, rs_sem, d, l, 0)
                rd.start()
                rs_fly[(d, l)] = rd

        for h in range(N_DEV - 1):
            r_ = (h + 1) % 2
            for l in range(LANES):
                for d in (0, 1):
                    rs_fly[(d, l)].wait()
                    if h < N_DEV - 2:
                        ck = recv_chunk(d, h)
                        rs_buf[d][r_, l] = (
                            rs_buf[d][r_, l]
                            + x_ref[rows(ck, d, l), :].astype(jnp.bfloat16))
                        rd = make_rdma(rs_buf, rs_sem, d, l, h + 1)
                        rd.start()
                        rs_fly[(d, l)] = rd
                    else:
                        my = rows(p, d, l)
                        y = (rs_buf[d][r_, l].astype(jnp.float32)
                             + x_ref[my, :] + resid_ref[my, :])
                        rms = jnp.sqrt(
                            jnp.mean(y * y, axis=-1, keepdims=True) + 1e-6)
                        o = y / rms * gamma_ref[:, :]
                        out_ref[my, :] = o
                        ag_buf[d][0, l] = o.astype(jnp.bfloat16)
                        rd = make_rdma(ag_buf, ag_sem, d, l, 0)
                        rd.start()
                        ag_fly[(d, l)] = rd

        for h in range(N_DEV - 1):
            r_ = (h + 1) % 2
            for l in range(LANES):
                for d in (0, 1):
                    ag_fly[(d, l)].wait()
                    if h < N_DEV - 2:
                        rd = make_rdma(ag_buf, ag_sem, d, l, h + 1)
                        rd.start()
                        ag_fly[(d, l)] = rd
                    org = ag_origin(d, h)
                    out_ref[rows(org, d, l), :] = (
                        ag_buf[d][r_, l].astype(jnp.float32))

    return pl.pallas_call(
        body,
        out_shape=jax.ShapeDtypeStruct((M, D), jnp.float32),
        in_specs=[
            pl.BlockSpec(memory_space=pltpu.VMEM),
            pl.BlockSpec(memory_space=pltpu.VMEM),
            pl.BlockSpec(memory_space=pltpu.VMEM),
        ],
        out_specs=pl.BlockSpec(memory_space=pltpu.VMEM),
        scratch_shapes=[
            pltpu.VMEM((2, LANES, S, D), jnp.bfloat16),
            pltpu.VMEM((2, LANES, S, D), jnp.bfloat16),
            pltpu.VMEM((2, LANES, S, D), jnp.bfloat16),
            pltpu.VMEM((2, LANES, S, D), jnp.bfloat16),
            pltpu.SemaphoreType.DMA((2, LANES)),
            pltpu.SemaphoreType.DMA((2, LANES)),
            pltpu.SemaphoreType.DMA((2, LANES)),
            pltpu.SemaphoreType.DMA((2, LANES)),
            pltpu.SemaphoreType.DMA((2, LANES)),
            pltpu.SemaphoreType.DMA((2, LANES)),
            pltpu.SemaphoreType.DMA((2, LANES)),
            pltpu.SemaphoreType.DMA((2, LANES)),
        ],
        compiler_params=pltpu.CompilerParams(
            collective_id=0, vmem_limit_bytes=100 * 1024 * 1024),
    )(x, resid, gamma2)
